# Initial kernel scaffold; baseline (speedup 1.0000x reference)
import jax
import jax.numpy as jnp
from jax.experimental import pallas as pl


def kernel(x, edge_index, batch, Ws, bs, gammas, betas):
    raise NotImplementedError("write your pallas kernel here")


# speedup vs baseline: 10.0631x; 10.0631x over previous
import functools

import jax
import jax.numpy as jnp
from jax import lax
from jax.experimental import pallas as pl
from jax.experimental.pallas import tpu as pltpu
from jax.experimental.pallas import tpu_sc as plsc

N = 10000
D = 128
L = 5
G = 64
E = 320000

NCORE = 2
NSUB = 16
NW = NCORE * NSUB
NP = 10240
EPT = E // NW
CH = 128
NCHUNK = (EPT + CH - 1) // CH
EPP = NCHUNK * CH
ROW_PAD = N
COL_PAD = NP - 1
RPT = NP // NSUB
NROWB = NP // 128


def _sc_mesh():
    return plsc.VectorSubcoreMesh(core_axis_name="c", subcore_axis_name="s",
                                  num_cores=NCORE, num_subcores=NSUB)


def _deg_body(col_hbm, ones_hbm, zeros_hbm, out_hbm, colv, onesv, acc):
    cid = lax.axis_index("c")
    sid = lax.axis_index("s")
    w = cid * NSUB + sid
    pltpu.sync_copy(col_hbm.at[w], colv)
    pltpu.sync_copy(ones_hbm, onesv)
    pltpu.sync_copy(zeros_hbm, acc.at[pl.ds(sid * RPT, RPT)])
    plsc.subcore_barrier()

    def body(j, carry):
        pltpu.sync_copy(onesv, acc.at[colv.at[j]], add=True)
        return carry

    lax.fori_loop(0, NCHUNK, body, 0)
    plsc.subcore_barrier()
    pltpu.sync_copy(acc.at[pl.ds(sid * RPT, RPT)],
                    out_hbm.at[cid, pl.ds(sid * RPT, RPT)])


def _scatter_body(s_hbm, row_hbm, col_hbm, zeros_hbm, out_hbm,
                  rowv, colv, gbuf, acc, gsem):
    cid = lax.axis_index("c")
    sid = lax.axis_index("s")
    w = cid * NSUB + sid
    pltpu.sync_copy(row_hbm.at[w], rowv)
    pltpu.sync_copy(col_hbm.at[w], colv)
    pltpu.sync_copy(zeros_hbm, acc.at[pl.ds(sid * RPT, RPT)])
    plsc.subcore_barrier()

    def body(j, carry):
        pltpu.async_copy(s_hbm.at[rowv.at[j]], gbuf, gsem).wait()
        pltpu.sync_copy(gbuf, acc.at[colv.at[j]], add=True)
        return carry

    lax.fori_loop(0, NCHUNK, body, 0)
    plsc.subcore_barrier()
    pltpu.sync_copy(acc.at[pl.ds(sid * RPT, RPT)],
                    out_hbm.at[cid, pl.ds(sid * RPT, RPT)])


@functools.cache
def _sc_kernels():
    deg = pl.kernel(
        _deg_body,
        out_type=jax.ShapeDtypeStruct((NCORE, NP, 1), jnp.float32),
        mesh=_sc_mesh(),
        scratch_types=[
            pltpu.VMEM((NCHUNK, CH), jnp.int32),
            pltpu.VMEM((CH, 1), jnp.float32),
            pltpu.VMEM_SHARED((NP, 1), jnp.float32),
        ],
    )
    scat = pl.kernel(
        _scatter_body,
        out_type=jax.ShapeDtypeStruct((NCORE, NP, D), jnp.float32),
        mesh=_sc_mesh(),
        scratch_types=[
            pltpu.VMEM((NCHUNK, CH), jnp.int32),
            pltpu.VMEM((NCHUNK, CH), jnp.int32),
            pltpu.VMEM((CH, D), jnp.float32),
            pltpu.VMEM_SHARED((NP, D), jnp.float32),
            pltpu.SemaphoreType.DMA,
        ],
    )
    return deg, scat


def _dinv_body(degp_ref, dinv_ref):
    deg = degp_ref[0] + degp_ref[1] + 1.0
    flat = (lax.broadcasted_iota(jnp.int32, (NROWB, 128), 0) * 128
            + lax.broadcasted_iota(jnp.int32, (NROWB, 128), 1))
    mask = (flat < N).astype(jnp.float32)
    dinv_ref[...] = lax.rsqrt(deg) * mask


def _s0_body(x_ref, w_ref, dinv_ref, s_ref):
    hl = jnp.dot(x_ref[...], w_ref[...], preferred_element_type=jnp.float32)
    s_ref[...] = hl * dinv_ref[...]


def _bn(conv):
    mean = jnp.sum(conv, axis=0, keepdims=True) * (1.0 / N)
    var = jnp.sum(conv * conv, axis=0, keepdims=True) * (1.0 / N) - mean * mean
    return mean, lax.rsqrt(var + 1e-5)


def _row_mask():
    return (lax.broadcasted_iota(jnp.int32, (NP, D), 0) < N).astype(jnp.float32)


def _layer_body(p_ref, s_ref, dinv_ref, b_ref, g_ref, be_ref, w_ref, out_ref):
    agg = p_ref[0] + p_ref[1] + s_ref[...]
    conv = (dinv_ref[...] * agg + b_ref[...]) * _row_mask()
    mean, rstd = _bn(conv)
    h = (conv - mean) * rstd * g_ref[...] + be_ref[...]
    h = jnp.maximum(h, 0.0)
    hl = jnp.dot(h, w_ref[...], preferred_element_type=jnp.float32)
    out_ref[...] = hl * dinv_ref[...]


def _final_body(p_ref, s_ref, dinv_ref, b_ref, g_ref, be_ref, batch_ref,
                h_ref, pool_ref):
    agg = p_ref[0] + p_ref[1] + s_ref[...]
    conv = (dinv_ref[...] * agg + b_ref[...]) * _row_mask()
    mean, rstd = _bn(conv)
    h = (conv - mean) * rstd * g_ref[...] + be_ref[...]
    h_ref[...] = h
    onehot = (lax.broadcasted_iota(jnp.int32, (G, NP), 0)
              == batch_ref[...]).astype(jnp.float32)
    pool_ref[...] = jnp.dot(onehot, h, preferred_element_type=jnp.float32)


def _tc(body, out_shape, *args):
    return pl.pallas_call(body, out_shape=out_shape)(*args)


def kernel(x, edge_index, batch, Ws, bs, gammas, betas):
    f32 = jnp.float32
    row = edge_index[0].astype(jnp.int32).reshape(NW, EPT)
    col = edge_index[1].astype(jnp.int32).reshape(NW, EPT)
    pad = EPP - EPT
    rowt = jnp.pad(row, ((0, 0), (0, pad)),
                   constant_values=ROW_PAD).reshape(NW, NCHUNK, CH)
    colt = jnp.pad(col, ((0, 0), (0, pad)),
                   constant_values=COL_PAD).reshape(NW, NCHUNK, CH)
    x_pad = jnp.pad(x.astype(f32), ((0, NP - N), (0, 0)))
    batch_pad = jnp.pad(batch.astype(jnp.int32), (0, NP - N),
                        constant_values=G).reshape(1, NP)
    ones1 = jnp.ones((CH, 1), f32)
    zeros1 = jnp.zeros((RPT, 1), f32)
    zerosD = jnp.zeros((RPT, D), f32)
    _deg_kernel, _scatter_kernel = _sc_kernels()

    degp = _deg_kernel(colt, ones1, zeros1)
    degp = degp.reshape(NCORE, NROWB, 128)
    dinv = _tc(_dinv_body, jax.ShapeDtypeStruct((NROWB, 128), f32), degp)
    dinv = dinv.reshape(NP, 1)

    s = _tc(_s0_body, jax.ShapeDtypeStruct((NP, D), f32),
            x_pad, Ws[0].astype(f32), dinv)
    for i in range(1, L):
        p = _scatter_kernel(s, rowt, colt, zerosD)
        s = _tc(_layer_body, jax.ShapeDtypeStruct((NP, D), f32),
                p, s, dinv,
                bs[i - 1].reshape(1, D).astype(f32),
                gammas[i - 1].reshape(1, D).astype(f32),
                betas[i - 1].reshape(1, D).astype(f32),
                Ws[i].astype(f32))
    p = _scatter_kernel(s, rowt, colt, zerosD)
    h, pool = _tc(_final_body,
                  (jax.ShapeDtypeStruct((NP, D), f32),
                   jax.ShapeDtypeStruct((G, D), f32)),
                  p, s, dinv,
                  bs[L - 1].reshape(1, D).astype(f32),
                  gammas[L - 1].reshape(1, D).astype(f32),
                  betas[L - 1].reshape(1, D).astype(f32),
                  batch_pad)
    return (pool, h[:N])
